# baseline (speedup 1.0000x reference)
import functools

import jax
import jax.numpy as jnp
from jax import lax
from jax.experimental import pallas as pl
from jax.experimental.pallas import tpu as pltpu
from jax.experimental.pallas import tpu_sc as plsc

N = 10000
D = 128
E = 320000
HD = 64
NP = 10112
RPS = NP // 16
ET = E + N
EP = 344064
TILES = 168
TB = 128
NCH = 21
TPC = TILES // NCH
DUMMY = NP - 1
CB = 8
NCB = RPS // CB


def _rsqrt16(x):
    i = lax.bitcast_convert_type(x, jnp.int32)
    i = jnp.int32(0x5F3759DF) - (i >> 1)
    y = lax.bitcast_convert_type(i, jnp.float32)
    for _ in range(3):
        y = y * (1.5 - 0.5 * x * y * y)
    return y


_mesh = plsc.VectorSubcoreMesh(core_axis_name="c", subcore_axis_name="s")


@functools.partial(
    pl.kernel,
    out_type=[
        jax.ShapeDtypeStruct((8 * NP, HD), jnp.float32),
        jax.ShapeDtypeStruct((2, 2 * NP, HD), jnp.float32),
    ],
    mesh=_mesh,
    compiler_params=pltpu.CompilerParams(use_tc_tiling_on_sc=False),
    scratch_types=[
        pltpu.VMEM_SHARED((NP, HD), jnp.float32),
        pltpu.VMEM_SHARED((NP, 16), jnp.float32),
        pltpu.VMEM((3, TB, HD), jnp.float32),
        pltpu.VMEM((CB, HD), jnp.float32),
        pltpu.VMEM((CB, HD), jnp.float32),
        pltpu.VMEM((CB, HD), jnp.float32),
        pltpu.VMEM((TPC, TB), jnp.int32),
        pltpu.VMEM((TPC, TB), jnp.int32),
        pltpu.VMEM((RPS, 16), jnp.float32),
        pltpu.VMEM((CB, 16), jnp.float32),
        pltpu.VMEM((TB, 16), jnp.float32),
        pltpu.VMEM((6, D), jnp.float32),
        pltpu.SemaphoreType.DMA((4,)),
        pltpu.SemaphoreType.DMA((4,)),
    ],
)
def _sc_mega(xrh, xih, colh, ridx, consts, ones16h, zrow, zr16,
             utab, w_all,
             acc, dacc, gbuf, bufA, bufB, bufC, colb, rowb,
             dis_l, dbuf, ones16, cbuf, gsem, ssem):
    cid = lax.axis_index("c")
    sid = lax.axis_index("s")
    base = sid * RPS
    coff = cid * NP

    pltpu.sync_copy(consts.at[cid], cbuf)
    pltpu.sync_copy(ones16h, ones16)
    pltpu.sync_copy(zrow, acc.at[pl.ds(base, RPS)])
    pltpu.sync_copy(zr16, dacc.at[pl.ds(base, RPS)])
    plsc.subcore_barrier()

    def dg_chunk(ch, _):
        pltpu.sync_copy(ridx.at[0, sid, pl.ds(ch * TPC, TPC)], rowb)

        def dg_body(i, _):
            r = i - 4

            @pl.when(r >= 0)
            def _():
                pltpu.make_async_copy(ones16, dacc.at[rowb.at[r]],
                                      ssem.at[lax.rem(r, 4)]).wait()

            @pl.when(i < TPC)
            def _():
                pltpu.async_copy(ones16, dacc.at[rowb.at[i]],
                                 ssem.at[lax.rem(i, 4)], add=True)
            return 0

        lax.fori_loop(0, TPC + 4, dg_body, 0)
        return 0

    lax.fori_loop(0, NCH, dg_chunk, 0)
    plsc.subcore_barrier()

    def dchunk(ch, _):
        off = ch * CB
        pltpu.sync_copy(dacc.at[pl.ds(base + off, CB)], dbuf)

        def rbody(r, _):
            v = jnp.maximum(dbuf[r], 1.0)
            dis_l[off + r] = _rsqrt16(v)
            return 0

        lax.fori_loop(0, CB, rbody, 0)
        return 0

    lax.fori_loop(0, NCB, dchunk, 0)

    for bf in range(2):
        def uchunk(ch, _):
            off = ch * CB
            pltpu.sync_copy(xrh.at[bf, pl.ds(base + off, CB)], bufA)
            pltpu.sync_copy(xih.at[bf, pl.ds(base + off, CB)], bufB)

            def bbody(r, _):
                dd = dis_l[off + r][0]
                for g in range(4):
                    sl = pl.ds(g * 16, 16)
                    cl = pl.ds(bf * HD + g * 16, 16)
                    bufC[r, sl] = dd * (cbuf[0, cl] * bufA[r, sl]
                                        + cbuf[1, cl] * bufB[r, sl])
                return 0

            lax.fori_loop(0, CB, bbody, 0)
            pltpu.sync_copy(
                bufC,
                utab.at[pl.ds((2 * cid + bf) * NP + base + off, CB)])
            return 0

        lax.fori_loop(0, NCB, uchunk, 0)
    plsc.subcore_barrier()

    def phase_q(q, _):
        h = q // 2
        f = lax.rem(q, 2)
        gp = 4 * h + 2 * cid + f

        def chunk(ch, _):
            pltpu.sync_copy(ridx.at[1, sid, pl.ds(ch * TPC, TPC)], rowb)
            pltpu.sync_copy(colh.at[gp, sid, pl.ds(ch * TPC, TPC)], colb)

            def body(i, _):
                r = i - 3

                @pl.when(jnp.logical_and(r >= 0, r < TPC))
                def _():
                    pr = lax.rem(r, 3)
                    pltpu.make_async_copy(gbuf.at[pr], acc.at[rowb.at[r]],
                                          ssem.at[pr]).wait()

                @pl.when(i < TPC)
                def _():
                    pltpu.async_copy(utab.at[colb.at[i]],
                                     gbuf.at[lax.rem(i, 3)],
                                     gsem.at[lax.rem(i, 3)])

                j = i - 1

                @pl.when(jnp.logical_and(j >= 0, j < TPC))
                def _():
                    pj = lax.rem(j, 3)
                    pltpu.make_async_copy(utab.at[colb.at[j]],
                                          gbuf.at[pj], gsem.at[pj]).wait()
                    pltpu.async_copy(gbuf.at[pj], acc.at[rowb.at[j]],
                                     ssem.at[pj], add=True)

                return 0

            lax.fori_loop(0, TPC + 3, body, 0)
            return 0

        lax.fori_loop(0, NCH, chunk, 0)
        plsc.subcore_barrier()

        @pl.when(h == 0)
        def _():
            def vchunk(ch, _):
                off = ch * CB
                pltpu.sync_copy(acc.at[pl.ds(base + off, CB)], bufA)

                def ubody(r, _):
                    di = dis_l[off + r][0]
                    d2 = di * di
                    for g in range(4):
                        sl = pl.ds(g * 16, 16)
                        bufC[r, sl] = d2 * bufA[r, sl]
                    return 0

                lax.fori_loop(0, CB, ubody, 0)
                pltpu.sync_copy(
                    bufC,
                    utab.at[pl.ds((4 + 2 * cid) * NP + f * NP
                                  + base + off, CB)])
                return 0

            lax.fori_loop(0, NCB, vchunk, 0)
            pltpu.sync_copy(zrow, acc.at[pl.ds(base, RPS)])

        @pl.when(h == 1)
        def _():
            def fchunk(ch, _):
                off = ch * CB
                pltpu.sync_copy(xrh.at[f, pl.ds(base + off, CB)], bufA)
                pltpu.sync_copy(xih.at[f, pl.ds(base + off, CB)], bufB)

                def fbody1(r, _):
                    for g in range(4):
                        sl = pl.ds(g * 16, 16)
                        cl = pl.ds(f * HD + g * 16, 16)
                        bufC[r, sl] = (cbuf[2, cl] * bufA[r, sl]
                                       + cbuf[3, cl] * bufB[r, sl])
                    return 0

                lax.fori_loop(0, CB, fbody1, 0)
                pltpu.sync_copy(
                    utab.at[pl.ds((4 + 2 * cid) * NP + f * NP
                                  + base + off, CB)],
                    bufA)
                pltpu.sync_copy(acc.at[pl.ds(base + off, CB)], bufB)

                def fbody2(r, _):
                    dv = dis_l[off + r]
                    di = dv[0]
                    sd = _rsqrt16(dv * dv)[0]
                    for g in range(4):
                        sl = pl.ds(g * 16, 16)
                        cl = pl.ds(f * HD + g * 16, 16)
                        bufC[r, sl] = (bufC[r, sl]
                                       + sd * (cbuf[4, cl] * bufA[r, sl])
                                       + di * (cbuf[5, cl] * bufB[r, sl]))
                    return 0

                lax.fori_loop(0, CB, fbody2, 0)
                pltpu.sync_copy(bufC,
                                w_all.at[f, pl.ds(coff + base + off, CB)])
                return 0

            lax.fori_loop(0, NCB, fchunk, 0)
            pltpu.sync_copy(zrow, acc.at[pl.ds(base, RPS)])

        plsc.subcore_barrier()
        return 0

    lax.fori_loop(0, 4, phase_q, 0)


def _tc_body(wr_ref, wi_ref, xr_ref, xi_ref, Wr_ref, Wi_ref, br_ref, bi_ref,
             or_ref, oi_ref):
    wr = wr_ref[...]
    wi = wi_ref[...]
    Wr = Wr_ref[...]
    Wi = Wi_ref[...]
    hp = jax.lax.Precision.HIGHEST
    or_ref[...] = (jnp.dot(wr, Wr, precision=hp,
                           preferred_element_type=jnp.float32)
                   - jnp.dot(wi, Wi, precision=hp,
                             preferred_element_type=jnp.float32)
                   + br_ref[...] + xr_ref[...])
    oi_ref[...] = (jnp.dot(wr, Wi, precision=hp,
                           preferred_element_type=jnp.float32)
                   + jnp.dot(wi, Wr, precision=hp,
                             preferred_element_type=jnp.float32)
                   + bi_ref[...] + xi_ref[...])


_TCB = 256
_tc_grid = (N + _TCB - 1) // _TCB


_row_spec = pl.BlockSpec((_TCB, D), lambda i: (i, 0))
_full_spec = pl.BlockSpec((D, D), lambda i: (0, 0))
_bias_spec = pl.BlockSpec((1, D), lambda i: (0, 0))

_tc_call = pl.pallas_call(
    _tc_body,
    grid=_tc_grid,
    in_specs=[_row_spec, _row_spec, _row_spec, _row_spec,
              _full_spec, _full_spec, _bias_spec, _bias_spec],
    out_specs=[_row_spec, _row_spec],
    out_shape=[jax.ShapeDtypeStruct((N, D), jnp.float32),
               jax.ShapeDtypeStruct((N, D), jnp.float32)],
)


def kernel(x_real, x_imag, edge_index, hop_weights, phase, gate, Wr, Wi, br, bi):
    f32 = jnp.float32
    xr = jnp.pad(x_real.astype(f32), ((0, NP - N), (0, 0)))
    xi = jnp.pad(x_imag.astype(f32), ((0, NP - N), (0, 0)))
    xrh = jnp.stack([xr[:, :HD], xr[:, HD:]])
    xih = jnp.stack([xi[:, :HD], xi[:, HD:]])

    ar = jnp.arange(N, dtype=jnp.int32)
    row = jnp.concatenate([edge_index[0].astype(jnp.int32), ar])
    col = jnp.concatenate([edge_index[1].astype(jnp.int32), ar])
    padv = jnp.full((EP - ET,), DUMMY, jnp.int32)
    rowp = jnp.concatenate([row, padv])
    colp = jnp.concatenate([col, padv])
    ridx = jnp.stack([colp, rowp]).reshape(2, 16, TILES, TB)
    colh = jnp.stack([colp + p * NP for p in range(8)]).reshape(
        8, 16, TILES, TB)

    c = jnp.cos(phase)
    s = jnp.sin(phase)
    ew = jax.nn.sigmoid(gate)
    hw = jax.nn.softmax(hop_weights)
    g1 = ew * hw[1]
    g2 = ew * hw[2]
    consts = jnp.stack([
        jnp.stack([c, -s, ew * hw[0] * c, -(ew * hw[0] * s), g1, g2]),
        jnp.stack([s, c, ew * hw[0] * s, ew * hw[0] * c, g1, g2]),
    ]).astype(f32)

    ones16h = jnp.ones((TB, 16), f32)
    zrow = jnp.zeros((RPS, HD), f32)
    zr16 = jnp.zeros((RPS, 16), f32)

    utab, w = _sc_mega(xrh, xih, colh, ridx, consts, ones16h, zrow, zr16)
    del utab
    wr = jnp.concatenate([w[0, :N], w[1, :N]], axis=1)
    wi = jnp.concatenate([w[0, NP:NP + N], w[1, NP:NP + N]], axis=1)

    out_r, out_i = _tc_call(wr, wi, x_real, x_imag, Wr, Wi,
                            br.reshape(1, D), bi.reshape(1, D))
    return (out_r, out_i)


# speedup vs baseline: 1.2116x; 1.2116x over previous
import functools

import jax
import jax.numpy as jnp
from jax import lax
from jax.experimental import pallas as pl
from jax.experimental.pallas import tpu as pltpu
from jax.experimental.pallas import tpu_sc as plsc

N = 10000
D = 128
E = 320000
HD = 64
NP = 10112
RPS = NP // 16
ET = E + N
EP = 344064
TILES = 168
TB = 128
NCH = 7
TPC = TILES // NCH
DUMMY = NP - 1
CB = 64
CHUNKS = tuple((k * CB, CB) for k in range(9)) + ((9 * CB, RPS - 9 * CB),)


def _rsqrt16(x):
    i = lax.bitcast_convert_type(x, jnp.int32)
    i = jnp.int32(0x5F3759DF) - (i >> 1)
    y = lax.bitcast_convert_type(i, jnp.float32)
    for _ in range(3):
        y = y * (1.5 - 0.5 * x * y * y)
    return y


_mesh = plsc.VectorSubcoreMesh(core_axis_name="c", subcore_axis_name="s")


@functools.partial(
    pl.kernel,
    out_type=[
        jax.ShapeDtypeStruct((8 * NP, HD), jnp.float32),
        jax.ShapeDtypeStruct((2, 2 * NP, HD), jnp.float32),
    ],
    mesh=_mesh,
    compiler_params=pltpu.CompilerParams(use_tc_tiling_on_sc=False),
    scratch_types=[
        pltpu.VMEM_SHARED((NP, HD), jnp.float32),
        pltpu.VMEM_SHARED((NP, 16), jnp.float32),
        pltpu.VMEM((4, TB, HD), jnp.float32),
        pltpu.VMEM((CB, HD), jnp.float32),
        pltpu.VMEM((CB, HD), jnp.float32),
        pltpu.VMEM((CB, HD), jnp.float32),
        pltpu.VMEM((TPC, TB), jnp.int32),
        pltpu.VMEM((TPC, TB), jnp.int32),
        pltpu.VMEM((RPS, 16), jnp.float32),
        pltpu.VMEM((CB, 16), jnp.float32),
        pltpu.VMEM((TB, 16), jnp.float32),
        pltpu.VMEM((6, D), jnp.float32),
        pltpu.SemaphoreType.DMA((4,)),
        pltpu.SemaphoreType.DMA((4,)),
    ],
)
def _sc_mega(xrh, xih, colh, ridx, consts, ones16h, zrow, zr16,
             utab, w_all,
             acc, dacc, gbuf, bufA, bufB, bufC, colb, rowb,
             dis_l, dbuf, ones16, cbuf, gsem, ssem):
    cid = lax.axis_index("c")
    sid = lax.axis_index("s")
    base = sid * RPS
    coff = cid * NP

    pltpu.sync_copy(consts.at[cid], cbuf)
    pltpu.sync_copy(ones16h, ones16)
    pltpu.sync_copy(zrow, acc.at[pl.ds(base, RPS)])
    pltpu.sync_copy(zr16, dacc.at[pl.ds(base, RPS)])
    plsc.subcore_barrier()

    def dg_chunk(ch, _):
        pltpu.sync_copy(ridx.at[0, sid, pl.ds(ch * TPC, TPC)], rowb)

        def dg_body(i, _):
            r = i - 4

            @pl.when(r >= 0)
            def _():
                pltpu.make_async_copy(ones16, dacc.at[rowb.at[r]],
                                      ssem.at[lax.rem(r, 4)]).wait()

            @pl.when(i < TPC)
            def _():
                pltpu.async_copy(ones16, dacc.at[rowb.at[i]],
                                 ssem.at[lax.rem(i, 4)], add=True)
            return 0

        lax.fori_loop(0, TPC + 4, dg_body, 0)
        return 0

    lax.fori_loop(0, NCH, dg_chunk, 0)
    plsc.subcore_barrier()

    for off, sz in CHUNKS:
        pltpu.sync_copy(dacc.at[pl.ds(base + off, sz)],
                        dbuf.at[pl.ds(0, sz)])

        def rbody(r, _):
            v = jnp.maximum(dbuf[r], 1.0)
            dis_l[off + r] = _rsqrt16(v)
            return 0

        lax.fori_loop(0, sz, rbody, 0)

    for bf in range(2):
        for off, sz in CHUNKS:
            pltpu.sync_copy(xrh.at[bf, pl.ds(base + off, sz)],
                            bufA.at[pl.ds(0, sz)])
            pltpu.sync_copy(xih.at[bf, pl.ds(base + off, sz)],
                            bufB.at[pl.ds(0, sz)])

            def bbody(r, _):
                dd = dis_l[off + r][0]
                for g in range(4):
                    sl = pl.ds(g * 16, 16)
                    cl = pl.ds(bf * HD + g * 16, 16)
                    bufC[r, sl] = dd * (cbuf[0, cl] * bufA[r, sl]
                                        + cbuf[1, cl] * bufB[r, sl])
                return 0

            lax.fori_loop(0, sz, bbody, 0)
            pltpu.sync_copy(
                bufC.at[pl.ds(0, sz)],
                utab.at[pl.ds((2 * cid + bf) * NP + base + off, sz)])
    plsc.subcore_barrier()

    def phase_q(q, _):
        h = q // 2
        f = lax.rem(q, 2)
        gp = 4 * h + 2 * cid + f

        def chunk(ch, _):
            pltpu.sync_copy(ridx.at[1, sid, pl.ds(ch * TPC, TPC)], rowb)
            pltpu.sync_copy(colh.at[gp, sid, pl.ds(ch * TPC, TPC)], colb)

            def body(i, _):
                r = i - 4

                @pl.when(jnp.logical_and(r >= 0, r < TPC))
                def _():
                    pr = lax.rem(r, 4)
                    pltpu.make_async_copy(gbuf.at[pr], acc.at[rowb.at[r]],
                                          ssem.at[pr]).wait()

                @pl.when(i < TPC)
                def _():
                    pltpu.async_copy(utab.at[colb.at[i]],
                                     gbuf.at[lax.rem(i, 4)],
                                     gsem.at[lax.rem(i, 4)])

                j = i - 1

                @pl.when(jnp.logical_and(j >= 0, j < TPC))
                def _():
                    pj = lax.rem(j, 4)
                    pltpu.make_async_copy(utab.at[colb.at[j]],
                                          gbuf.at[pj], gsem.at[pj]).wait()
                    pltpu.async_copy(gbuf.at[pj], acc.at[rowb.at[j]],
                                     ssem.at[pj], add=True)

                return 0

            lax.fori_loop(0, TPC + 4, body, 0)
            return 0

        lax.fori_loop(0, NCH, chunk, 0)
        plsc.subcore_barrier()

        @pl.when(h == 0)
        def _():
            for off, sz in CHUNKS:
                pltpu.sync_copy(acc.at[pl.ds(base + off, sz)],
                                bufA.at[pl.ds(0, sz)])

                def ubody(r, _):
                    di = dis_l[off + r][0]
                    d2 = di * di
                    for g in range(4):
                        sl = pl.ds(g * 16, 16)
                        bufC[r, sl] = d2 * bufA[r, sl]
                    return 0

                lax.fori_loop(0, sz, ubody, 0)
                pltpu.sync_copy(
                    bufC.at[pl.ds(0, sz)],
                    utab.at[pl.ds((4 + 2 * cid + f) * NP + base + off, sz)])
            pltpu.sync_copy(zrow, acc.at[pl.ds(base, RPS)])

        @pl.when(h == 1)
        def _():
            for off, sz in CHUNKS:
                pltpu.sync_copy(xrh.at[f, pl.ds(base + off, sz)],
                                bufA.at[pl.ds(0, sz)])
                pltpu.sync_copy(xih.at[f, pl.ds(base + off, sz)],
                                bufB.at[pl.ds(0, sz)])

                def fbody1(r, _):
                    for g in range(4):
                        sl = pl.ds(g * 16, 16)
                        cl = pl.ds(f * HD + g * 16, 16)
                        bufC[r, sl] = (cbuf[2, cl] * bufA[r, sl]
                                       + cbuf[3, cl] * bufB[r, sl])
                    return 0

                lax.fori_loop(0, sz, fbody1, 0)
                pltpu.sync_copy(
                    utab.at[pl.ds((4 + 2 * cid) * NP + f * NP
                                  + base + off, sz)],
                    bufA.at[pl.ds(0, sz)])
                pltpu.sync_copy(acc.at[pl.ds(base + off, sz)],
                                bufB.at[pl.ds(0, sz)])

                def fbody2(r, _):
                    dv = dis_l[off + r]
                    di = dv[0]
                    sd = _rsqrt16(dv * dv)[0]
                    for g in range(4):
                        sl = pl.ds(g * 16, 16)
                        cl = pl.ds(f * HD + g * 16, 16)
                        bufC[r, sl] = (bufC[r, sl]
                                       + sd * (cbuf[4, cl] * bufA[r, sl])
                                       + di * (cbuf[5, cl] * bufB[r, sl]))
                    return 0

                lax.fori_loop(0, sz, fbody2, 0)
                pltpu.sync_copy(bufC.at[pl.ds(0, sz)],
                                w_all.at[f, pl.ds(coff + base + off, sz)])
            pltpu.sync_copy(zrow, acc.at[pl.ds(base, RPS)])

        plsc.subcore_barrier()
        return 0

    lax.fori_loop(0, 4, phase_q, 0)


def _tc_body(wr_ref, wi_ref, xr_ref, xi_ref, Wr_ref, Wi_ref, br_ref, bi_ref,
             or_ref, oi_ref):
    wr = wr_ref[...]
    wi = wi_ref[...]
    Wr = Wr_ref[...]
    Wi = Wi_ref[...]
    hp = jax.lax.Precision.HIGHEST
    or_ref[...] = (jnp.dot(wr, Wr, precision=hp,
                           preferred_element_type=jnp.float32)
                   - jnp.dot(wi, Wi, precision=hp,
                             preferred_element_type=jnp.float32)
                   + br_ref[...] + xr_ref[...])
    oi_ref[...] = (jnp.dot(wr, Wi, precision=hp,
                           preferred_element_type=jnp.float32)
                   + jnp.dot(wi, Wr, precision=hp,
                             preferred_element_type=jnp.float32)
                   + bi_ref[...] + xi_ref[...])


_TCB = 256
_tc_grid = (N + _TCB - 1) // _TCB


_row_spec = pl.BlockSpec((_TCB, D), lambda i: (i, 0))
_full_spec = pl.BlockSpec((D, D), lambda i: (0, 0))
_bias_spec = pl.BlockSpec((1, D), lambda i: (0, 0))

_tc_call = pl.pallas_call(
    _tc_body,
    grid=_tc_grid,
    in_specs=[_row_spec, _row_spec, _row_spec, _row_spec,
              _full_spec, _full_spec, _bias_spec, _bias_spec],
    out_specs=[_row_spec, _row_spec],
    out_shape=[jax.ShapeDtypeStruct((N, D), jnp.float32),
               jax.ShapeDtypeStruct((N, D), jnp.float32)],
)


def kernel(x_real, x_imag, edge_index, hop_weights, phase, gate, Wr, Wi, br, bi):
    f32 = jnp.float32
    xr = jnp.pad(x_real.astype(f32), ((0, NP - N), (0, 0)))
    xi = jnp.pad(x_imag.astype(f32), ((0, NP - N), (0, 0)))
    xrh = jnp.stack([xr[:, :HD], xr[:, HD:]])
    xih = jnp.stack([xi[:, :HD], xi[:, HD:]])

    ar = jnp.arange(N, dtype=jnp.int32)
    row = jnp.concatenate([edge_index[0].astype(jnp.int32), ar])
    col = jnp.concatenate([edge_index[1].astype(jnp.int32), ar])
    padv = jnp.full((EP - ET,), DUMMY, jnp.int32)
    rowp = jnp.concatenate([row, padv])
    colp = jnp.concatenate([col, padv])
    ridx = jnp.stack([colp, rowp]).reshape(2, 16, TILES, TB)
    colh = jnp.stack([colp + p * NP for p in range(8)]).reshape(
        8, 16, TILES, TB)

    c = jnp.cos(phase)
    s = jnp.sin(phase)
    ew = jax.nn.sigmoid(gate)
    hw = jax.nn.softmax(hop_weights)
    g1 = ew * hw[1]
    g2 = ew * hw[2]
    consts = jnp.stack([
        jnp.stack([c, -s, ew * hw[0] * c, -(ew * hw[0] * s), g1, g2]),
        jnp.stack([s, c, ew * hw[0] * s, ew * hw[0] * c, g1, g2]),
    ]).astype(f32)

    ones16h = jnp.ones((TB, 16), f32)
    zrow = jnp.zeros((RPS, HD), f32)
    zr16 = jnp.zeros((RPS, 16), f32)

    utab, w = _sc_mega(xrh, xih, colh, ridx, consts, ones16h, zrow, zr16)
    del utab
    wr = jnp.concatenate([w[0, :N], w[1, :N]], axis=1)
    wi = jnp.concatenate([w[0, NP:NP + N], w[1, NP:NP + N]], axis=1)

    out_r, out_i = _tc_call(wr, wi, x_real, x_imag, Wr, Wi,
                            br.reshape(1, D), bi.reshape(1, D))
    return (out_r, out_i)


# speedup vs baseline: 1.6024x; 1.3225x over previous
import functools

import jax
import jax.numpy as jnp
from jax import lax
from jax.experimental import pallas as pl
from jax.experimental.pallas import tpu as pltpu
from jax.experimental.pallas import tpu_sc as plsc

N = 10000
D = 128
E = 320000
HD = 64
NP = 10112
RPS = NP // 16
EP = 327680
TILES = 160
TB = 128
NCH = 10
TPC = TILES // NCH
DUMMY = NP - 1
CB = 64
CHUNKS = tuple((k * CB, CB) for k in range(9)) + ((9 * CB, RPS - 9 * CB),)


def _rsqrt16(x):
    i = lax.bitcast_convert_type(x, jnp.int32)
    i = jnp.int32(0x5F3759DF) - (i >> 1)
    y = lax.bitcast_convert_type(i, jnp.float32)
    for _ in range(3):
        y = y * (1.5 - 0.5 * x * y * y)
    return y


_mesh = plsc.VectorSubcoreMesh(core_axis_name="c", subcore_axis_name="s")


@functools.partial(
    pl.kernel,
    out_type=[
        jax.ShapeDtypeStruct((8 * NP, HD), jnp.float32),
        jax.ShapeDtypeStruct((2, 2 * NP, HD), jnp.float32),
    ],
    mesh=_mesh,
    compiler_params=pltpu.CompilerParams(use_tc_tiling_on_sc=False),
    scratch_types=[
        pltpu.VMEM_SHARED((NP, HD), jnp.float32),
        pltpu.VMEM_SHARED((NP, 16), jnp.float32),
        pltpu.VMEM((4, TB, HD), jnp.float32),
        pltpu.VMEM((CB, HD), jnp.float32),
        pltpu.VMEM((CB, HD), jnp.float32),
        pltpu.VMEM((CB, HD), jnp.float32),
        pltpu.VMEM((TPC, TB), jnp.int32),
        pltpu.VMEM((TPC, TB), jnp.int32),
        pltpu.VMEM((RPS, 16), jnp.float32),
        pltpu.VMEM((CB, 16), jnp.float32),
        pltpu.VMEM((TB, 16), jnp.float32),
        pltpu.VMEM((6, D), jnp.float32),
        pltpu.SemaphoreType.DMA((4,)),
        pltpu.SemaphoreType.DMA((4,)),
    ],
)
def _sc_mega(xrh, xih, colh, ridx, consts, ones16h, zrow, zr16,
             utab, w_all,
             acc, dacc, gbuf, bufA, bufB, bufC, colb, rowb,
             dis_l, dbuf, ones16, cbuf, gsem, ssem):
    cid = lax.axis_index("c")
    sid = lax.axis_index("s")
    base = sid * RPS
    coff = cid * NP

    pltpu.sync_copy(consts.at[cid], cbuf)
    pltpu.sync_copy(ones16h, ones16)
    pltpu.sync_copy(zrow, acc.at[pl.ds(base, RPS)])
    pltpu.sync_copy(zr16, dacc.at[pl.ds(base, RPS)])
    plsc.subcore_barrier()

    def dg_chunk(ch, _):
        pltpu.sync_copy(ridx.at[0, sid, pl.ds(ch * TPC, TPC)], rowb)

        def dg_body(i, _):
            r = i - 4

            @pl.when(r >= 0)
            def _():
                pltpu.make_async_copy(ones16, dacc.at[rowb.at[r]],
                                      ssem.at[lax.rem(r, 4)]).wait()

            @pl.when(i < TPC)
            def _():
                pltpu.async_copy(ones16, dacc.at[rowb.at[i]],
                                 ssem.at[lax.rem(i, 4)], add=True)
            return 0

        lax.fori_loop(0, TPC + 4, dg_body, 0)
        return 0

    lax.fori_loop(0, NCH, dg_chunk, 0)
    plsc.subcore_barrier()

    for off, sz in CHUNKS:
        pltpu.sync_copy(dacc.at[pl.ds(base + off, sz)],
                        dbuf.at[pl.ds(0, sz)])

        def rbody(r, _):
            v = dbuf[r] + 1.0
            dis_l[off + r] = _rsqrt16(v)
            return 0

        lax.fori_loop(0, sz, rbody, 0)

    for bf in range(2):
        for off, sz in CHUNKS:
            pltpu.sync_copy(xrh.at[bf, pl.ds(base + off, sz)],
                            bufA.at[pl.ds(0, sz)])
            pltpu.sync_copy(xih.at[bf, pl.ds(base + off, sz)],
                            bufB.at[pl.ds(0, sz)])

            def bbody(r, _):
                dd = dis_l[off + r][0]
                for g in range(4):
                    sl = pl.ds(g * 16, 16)
                    cl = pl.ds(bf * HD + g * 16, 16)
                    bufC[r, sl] = dd * (cbuf[0, cl] * bufA[r, sl]
                                        + cbuf[1, cl] * bufB[r, sl])
                return 0

            lax.fori_loop(0, sz, bbody, 0)
            pltpu.sync_copy(
                bufC.at[pl.ds(0, sz)],
                utab.at[pl.ds((2 * cid + bf) * NP + base + off, sz)])
    plsc.subcore_barrier()

    def phase_q(q, _):
        h = q // 2
        f = lax.rem(q, 2)
        gp = 4 * h + 2 * cid + f

        def chunk(ch, _):
            pltpu.sync_copy(ridx.at[1, sid, pl.ds(ch * TPC, TPC)], rowb)
            pltpu.sync_copy(colh.at[gp, sid, pl.ds(ch * TPC, TPC)], colb)

            def body(i, _):
                r = i - 4

                @pl.when(jnp.logical_and(r >= 0, r < TPC))
                def _():
                    pr = lax.rem(r, 4)
                    pltpu.make_async_copy(gbuf.at[pr], acc.at[rowb.at[r]],
                                          ssem.at[pr]).wait()

                @pl.when(i < TPC)
                def _():
                    pltpu.async_copy(utab.at[colb.at[i]],
                                     gbuf.at[lax.rem(i, 4)],
                                     gsem.at[lax.rem(i, 4)])

                j = i - 1

                @pl.when(jnp.logical_and(j >= 0, j < TPC))
                def _():
                    pj = lax.rem(j, 4)
                    pltpu.make_async_copy(utab.at[colb.at[j]],
                                          gbuf.at[pj], gsem.at[pj]).wait()
                    pltpu.async_copy(gbuf.at[pj], acc.at[rowb.at[j]],
                                     ssem.at[pj], add=True)

                return 0

            lax.fori_loop(0, TPC + 4, body, 0)
            return 0

        lax.fori_loop(0, NCH, chunk, 0)
        plsc.subcore_barrier()

        @pl.when(h == 0)
        def _():
            for off, sz in CHUNKS:
                pltpu.sync_copy(acc.at[pl.ds(base + off, sz)],
                                bufA.at[pl.ds(0, sz)])
                pltpu.sync_copy(
                    utab.at[pl.ds((2 * cid + f) * NP + base + off, sz)],
                    bufB.at[pl.ds(0, sz)])

                def ubody(r, _):
                    di = dis_l[off + r][0]
                    d2 = di * di
                    for g in range(4):
                        sl = pl.ds(g * 16, 16)
                        bufC[r, sl] = d2 * (bufA[r, sl] + bufB[r, sl])
                    return 0

                lax.fori_loop(0, sz, ubody, 0)
                pltpu.sync_copy(
                    bufC.at[pl.ds(0, sz)],
                    utab.at[pl.ds((4 + 2 * cid + f) * NP + base + off, sz)])
            pltpu.sync_copy(zrow, acc.at[pl.ds(base, RPS)])

        @pl.when(h == 1)
        def _():
            for off, sz in CHUNKS:
                pltpu.sync_copy(xrh.at[f, pl.ds(base + off, sz)],
                                bufA.at[pl.ds(0, sz)])
                pltpu.sync_copy(xih.at[f, pl.ds(base + off, sz)],
                                bufB.at[pl.ds(0, sz)])

                def fbody1(r, _):
                    for g in range(4):
                        sl = pl.ds(g * 16, 16)
                        cl = pl.ds(f * HD + g * 16, 16)
                        bufC[r, sl] = (cbuf[2, cl] * bufA[r, sl]
                                       + cbuf[3, cl] * bufB[r, sl])
                    return 0

                lax.fori_loop(0, sz, fbody1, 0)
                pltpu.sync_copy(
                    utab.at[pl.ds((4 + 2 * cid) * NP + f * NP
                                  + base + off, sz)],
                    bufA.at[pl.ds(0, sz)])
                pltpu.sync_copy(acc.at[pl.ds(base + off, sz)],
                                bufB.at[pl.ds(0, sz)])

                def fbody2(r, _):
                    dv = dis_l[off + r]
                    di = dv[0]
                    sd = _rsqrt16(dv * dv)[0]
                    for g in range(4):
                        sl = pl.ds(g * 16, 16)
                        cl = pl.ds(f * HD + g * 16, 16)
                        bufC[r, sl] = (bufC[r, sl]
                                       + sd * (cbuf[4, cl] * bufA[r, sl])
                                       + di * (cbuf[5, cl]
                                               * (bufA[r, sl] + bufB[r, sl])))
                    return 0

                lax.fori_loop(0, sz, fbody2, 0)
                pltpu.sync_copy(bufC.at[pl.ds(0, sz)],
                                w_all.at[f, pl.ds(coff + base + off, sz)])
            pltpu.sync_copy(zrow, acc.at[pl.ds(base, RPS)])

        plsc.subcore_barrier()
        return 0

    lax.fori_loop(0, 4, phase_q, 0)


def _tc_body(wr_ref, wi_ref, xr_ref, xi_ref, Wr_ref, Wi_ref, br_ref, bi_ref,
             or_ref, oi_ref):
    wr = wr_ref[...]
    wi = wi_ref[...]
    Wr = Wr_ref[...]
    Wi = Wi_ref[...]
    hp = jax.lax.Precision.HIGHEST
    or_ref[...] = (jnp.dot(wr, Wr, precision=hp,
                           preferred_element_type=jnp.float32)
                   - jnp.dot(wi, Wi, precision=hp,
                             preferred_element_type=jnp.float32)
                   + br_ref[...] + xr_ref[...])
    oi_ref[...] = (jnp.dot(wr, Wi, precision=hp,
                           preferred_element_type=jnp.float32)
                   + jnp.dot(wi, Wr, precision=hp,
                             preferred_element_type=jnp.float32)
                   + bi_ref[...] + xi_ref[...])


_TCB = 256
_tc_grid = (N + _TCB - 1) // _TCB


_row_spec = pl.BlockSpec((_TCB, D), lambda i: (i, 0))
_full_spec = pl.BlockSpec((D, D), lambda i: (0, 0))
_bias_spec = pl.BlockSpec((1, D), lambda i: (0, 0))

_tc_call = pl.pallas_call(
    _tc_body,
    grid=_tc_grid,
    in_specs=[_row_spec, _row_spec, _row_spec, _row_spec,
              _full_spec, _full_spec, _bias_spec, _bias_spec],
    out_specs=[_row_spec, _row_spec],
    out_shape=[jax.ShapeDtypeStruct((N, D), jnp.float32),
               jax.ShapeDtypeStruct((N, D), jnp.float32)],
)


def kernel(x_real, x_imag, edge_index, hop_weights, phase, gate, Wr, Wi, br, bi):
    f32 = jnp.float32
    xr = jnp.pad(x_real.astype(f32), ((0, NP - N), (0, 0)))
    xi = jnp.pad(x_imag.astype(f32), ((0, NP - N), (0, 0)))
    xrh = jnp.stack([xr[:, :HD], xr[:, HD:]])
    xih = jnp.stack([xi[:, :HD], xi[:, HD:]])

    padv = jnp.full((EP - E,), DUMMY, jnp.int32)
    rowp = jnp.concatenate([edge_index[0].astype(jnp.int32), padv])
    colp = jnp.concatenate([edge_index[1].astype(jnp.int32), padv])
    ridx = jnp.stack([colp, rowp]).reshape(2, 16, TILES, TB)
    colh = jnp.stack([colp + p * NP for p in range(8)]).reshape(
        8, 16, TILES, TB)

    c = jnp.cos(phase)
    s = jnp.sin(phase)
    ew = jax.nn.sigmoid(gate)
    hw = jax.nn.softmax(hop_weights)
    g1 = ew * hw[1]
    g2 = ew * hw[2]
    consts = jnp.stack([
        jnp.stack([c, -s, ew * hw[0] * c, -(ew * hw[0] * s), g1, g2]),
        jnp.stack([s, c, ew * hw[0] * s, ew * hw[0] * c, g1, g2]),
    ]).astype(f32)

    ones16h = jnp.ones((TB, 16), f32)
    zrow = jnp.zeros((RPS, HD), f32)
    zr16 = jnp.zeros((RPS, 16), f32)

    utab, w = _sc_mega(xrh, xih, colh, ridx, consts, ones16h, zrow, zr16)
    del utab
    wr = jnp.concatenate([w[0, :N], w[1, :N]], axis=1)
    wi = jnp.concatenate([w[0, NP:NP + N], w[1, NP:NP + N]], axis=1)

    out_r, out_i = _tc_call(wr, wi, x_real, x_imag, Wr, Wi,
                            br.reshape(1, D), bi.reshape(1, D))
    return (out_r, out_i)


# speedup vs baseline: 2.7796x; 1.7346x over previous
import functools

import jax
import jax.numpy as jnp
from jax import lax
from jax.experimental import pallas as pl
from jax.experimental.pallas import tpu as pltpu
from jax.experimental.pallas import tpu_sc as plsc

N = 10000
D = 128
E = 320000
HD = 64
NP = 10112
RPS = NP // 16
EP = 327680
TILES = 160
TB = 128
NCH = 10
TPC = TILES // NCH
DUMMY = NP - 1
CB = 64
CHUNKS = tuple((k * CB, CB) for k in range(9)) + ((9 * CB, RPS - 9 * CB),)


def _rsqrt16(x):
    i = lax.bitcast_convert_type(x, jnp.int32)
    i = jnp.int32(0x5F3759DF) - (i >> 1)
    y = lax.bitcast_convert_type(i, jnp.float32)
    for _ in range(3):
        y = y * (1.5 - 0.5 * x * y * y)
    return y


_mesh = plsc.VectorSubcoreMesh(core_axis_name="c", subcore_axis_name="s")


@functools.partial(
    pl.kernel,
    out_type=[
        jax.ShapeDtypeStruct((8 * NP, HD), jnp.float32),
        jax.ShapeDtypeStruct((2, 2 * NP, HD), jnp.float32),
    ],
    mesh=_mesh,
    compiler_params=pltpu.CompilerParams(use_tc_tiling_on_sc=False),
    scratch_types=[
        pltpu.VMEM_SHARED((NP, HD), jnp.float32),
        pltpu.VMEM_SHARED((NP, 16), jnp.float32),
        pltpu.VMEM((4, TB, HD), jnp.float32),
        pltpu.VMEM((CB, HD), jnp.float32),
        pltpu.VMEM((CB, HD), jnp.float32),
        pltpu.VMEM((CB, HD), jnp.float32),
        pltpu.VMEM((TPC, TB), jnp.int32),
        pltpu.VMEM((TPC, TB), jnp.int32),
        pltpu.VMEM((RPS, 16), jnp.float32),
        pltpu.VMEM((CB, 16), jnp.float32),
        pltpu.VMEM((TB, 16), jnp.float32),
        pltpu.VMEM((6, D), jnp.float32),
        pltpu.SemaphoreType.DMA((4,)),
        pltpu.SemaphoreType.DMA((4,)),
    ],
)
def _sc_mega(xrh, xih, colh, ridx, consts, ones16h, zrow, zr16,
             utab, w_all,
             acc, dacc, gbuf, bufA, bufB, bufC, colb, rowb,
             dis_l, dbuf, ones16, cbuf, gsem, ssem):
    cid = lax.axis_index("c")
    sid = lax.axis_index("s")
    base = sid * RPS
    coff = cid * NP

    pltpu.sync_copy(consts.at[cid], cbuf)
    pltpu.sync_copy(ones16h, ones16)
    pltpu.sync_copy(zrow, acc.at[pl.ds(base, RPS)])
    pltpu.sync_copy(zr16, dacc.at[pl.ds(base, RPS)])
    plsc.subcore_barrier()

    def dg_chunk(ch, _):
        pltpu.sync_copy(ridx.at[0, sid, pl.ds(ch * TPC, TPC)], rowb)

        def dg_body(i, _):
            r = i - 4

            @pl.when(r >= 0)
            def _():
                pltpu.make_async_copy(ones16, dacc.at[rowb.at[r]],
                                      ssem.at[lax.rem(r, 4)]).wait()

            @pl.when(i < TPC)
            def _():
                pltpu.async_copy(ones16, dacc.at[rowb.at[i]],
                                 ssem.at[lax.rem(i, 4)], add=True)
            return 0

        lax.fori_loop(0, TPC + 4, dg_body, 0)
        return 0

    lax.fori_loop(0, NCH, dg_chunk, 0)
    plsc.subcore_barrier()

    for off, sz in CHUNKS:
        pltpu.sync_copy(dacc.at[pl.ds(base + off, sz)],
                        dbuf.at[pl.ds(0, sz)])

        def rbody(r, _):
            v = dbuf[r] + 1.0
            dis_l[off + r] = _rsqrt16(v)
            return 0

        lax.fori_loop(0, sz, rbody, 0)

    for bf in range(2):
        for off, sz in CHUNKS:
            pltpu.sync_copy(xrh.at[bf, pl.ds(base + off, sz)],
                            bufA.at[pl.ds(0, sz)])
            pltpu.sync_copy(xih.at[bf, pl.ds(base + off, sz)],
                            bufB.at[pl.ds(0, sz)])

            def bbody(r, _):
                dd = dis_l[off + r][0]
                for g in range(4):
                    sl = pl.ds(g * 16, 16)
                    cl = pl.ds(bf * HD + g * 16, 16)
                    bufC[r, sl] = dd * (cbuf[0, cl] * bufA[r, sl]
                                        + cbuf[1, cl] * bufB[r, sl])
                return 0

            lax.fori_loop(0, sz, bbody, 0)
            pltpu.sync_copy(
                bufC.at[pl.ds(0, sz)],
                utab.at[pl.ds((2 * cid + bf) * NP + base + off, sz)])
    plsc.subcore_barrier()

    def phase_q(q, _):
        h = q // 2
        f = lax.rem(q, 2)
        gp = 4 * h + 2 * cid + f

        def chunk(ch, _):
            pltpu.sync_copy(ridx.at[1, sid, pl.ds(ch * TPC, TPC)], rowb)
            pltpu.sync_copy(colh.at[gp, sid, pl.ds(ch * TPC, TPC)], colb)

            def body(i, _):
                r = i - 4

                @pl.when(jnp.logical_and(r >= 0, r < TPC))
                def _():
                    pr = lax.rem(r, 4)
                    pltpu.make_async_copy(gbuf.at[pr], acc.at[rowb.at[r]],
                                          ssem.at[pr]).wait()

                @pl.when(i < TPC)
                def _():
                    pltpu.async_copy(utab.at[colb.at[i]],
                                     gbuf.at[lax.rem(i, 4)],
                                     gsem.at[lax.rem(i, 4)])

                j = i - 1

                @pl.when(jnp.logical_and(j >= 0, j < TPC))
                def _():
                    pj = lax.rem(j, 4)
                    pltpu.make_async_copy(utab.at[colb.at[j]],
                                          gbuf.at[pj], gsem.at[pj]).wait()
                    pltpu.async_copy(gbuf.at[pj], acc.at[rowb.at[j]],
                                     ssem.at[pj], add=True)

                return 0

            lax.fori_loop(0, TPC + 4, body, 0)
            return 0

        lax.fori_loop(0, NCH, chunk, 0)
        plsc.subcore_barrier()

        @pl.when(h == 0)
        def _():
            for off, sz in CHUNKS:
                pltpu.sync_copy(acc.at[pl.ds(base + off, sz)],
                                bufA.at[pl.ds(0, sz)])
                pltpu.sync_copy(
                    utab.at[pl.ds((2 * cid + f) * NP + base + off, sz)],
                    bufB.at[pl.ds(0, sz)])

                def ubody(r, _):
                    di = dis_l[off + r][0]
                    d2 = di * di
                    for g in range(4):
                        sl = pl.ds(g * 16, 16)
                        bufC[r, sl] = d2 * (bufA[r, sl] + bufB[r, sl])
                    return 0

                lax.fori_loop(0, sz, ubody, 0)
                pltpu.sync_copy(
                    bufC.at[pl.ds(0, sz)],
                    utab.at[pl.ds((4 + 2 * cid + f) * NP + base + off, sz)])
            pltpu.sync_copy(zrow, acc.at[pl.ds(base, RPS)])

        @pl.when(h == 1)
        def _():
            for off, sz in CHUNKS:
                pltpu.sync_copy(xrh.at[f, pl.ds(base + off, sz)],
                                bufA.at[pl.ds(0, sz)])
                pltpu.sync_copy(xih.at[f, pl.ds(base + off, sz)],
                                bufB.at[pl.ds(0, sz)])

                def fbody1(r, _):
                    for g in range(4):
                        sl = pl.ds(g * 16, 16)
                        cl = pl.ds(f * HD + g * 16, 16)
                        bufC[r, sl] = (cbuf[2, cl] * bufA[r, sl]
                                       + cbuf[3, cl] * bufB[r, sl])
                    return 0

                lax.fori_loop(0, sz, fbody1, 0)
                pltpu.sync_copy(
                    utab.at[pl.ds((4 + 2 * cid) * NP + f * NP
                                  + base + off, sz)],
                    bufA.at[pl.ds(0, sz)])
                pltpu.sync_copy(acc.at[pl.ds(base + off, sz)],
                                bufB.at[pl.ds(0, sz)])

                def fbody2(r, _):
                    dv = dis_l[off + r]
                    di = dv[0]
                    sd = _rsqrt16(dv * dv)[0]
                    for g in range(4):
                        sl = pl.ds(g * 16, 16)
                        cl = pl.ds(f * HD + g * 16, 16)
                        bufC[r, sl] = (bufC[r, sl]
                                       + sd * (cbuf[4, cl] * bufA[r, sl])
                                       + di * (cbuf[5, cl]
                                               * (bufA[r, sl] + bufB[r, sl])))
                    return 0

                lax.fori_loop(0, sz, fbody2, 0)
                pltpu.sync_copy(bufC.at[pl.ds(0, sz)],
                                w_all.at[f, pl.ds(coff + base + off, sz)])
            pltpu.sync_copy(zrow, acc.at[pl.ds(base, RPS)])

        plsc.subcore_barrier()
        return 0

    lax.fori_loop(0, 4, phase_q, 0)


def _tc_body(wr_ref, wi_ref, xr_ref, xi_ref, Wr_ref, Wi_ref, br_ref, bi_ref,
             or_ref, oi_ref):
    wr = wr_ref[...]
    wi = wi_ref[...]
    Wr = Wr_ref[...]
    Wi = Wi_ref[...]
    hp = jax.lax.Precision.HIGHEST
    or_ref[...] = (jnp.dot(wr, Wr, precision=hp,
                           preferred_element_type=jnp.float32)
                   - jnp.dot(wi, Wi, precision=hp,
                             preferred_element_type=jnp.float32)
                   + br_ref[...] + xr_ref[...])
    oi_ref[...] = (jnp.dot(wr, Wi, precision=hp,
                           preferred_element_type=jnp.float32)
                   + jnp.dot(wi, Wr, precision=hp,
                             preferred_element_type=jnp.float32)
                   + bi_ref[...] + xi_ref[...])


_TCB = 256
_tc_grid = (N + _TCB - 1) // _TCB


_row_spec = pl.BlockSpec((_TCB, D), lambda i: (i, 0))
_full_spec = pl.BlockSpec((D, D), lambda i: (0, 0))
_bias_spec = pl.BlockSpec((1, D), lambda i: (0, 0))

_tc_call = pl.pallas_call(
    _tc_body,
    grid=_tc_grid,
    in_specs=[_row_spec, _row_spec, _row_spec, _row_spec,
              _full_spec, _full_spec, _bias_spec, _bias_spec],
    out_specs=[_row_spec, _row_spec],
    out_shape=[jax.ShapeDtypeStruct((N, D), jnp.float32),
               jax.ShapeDtypeStruct((N, D), jnp.float32)],
)


def kernel(x_real, x_imag, edge_index, hop_weights, phase, gate, Wr, Wi, br, bi):
    f32 = jnp.float32
    xr = jnp.pad(x_real.astype(f32), ((0, NP - N), (0, 0)))
    xi = jnp.pad(x_imag.astype(f32), ((0, NP - N), (0, 0)))
    xrh = jnp.stack([xr[:, :HD], xr[:, HD:]])
    xih = jnp.stack([xi[:, :HD], xi[:, HD:]])

    padv = N + jnp.arange(EP - E, dtype=jnp.int32) % (NP - N)
    rowp = jnp.concatenate([edge_index[0].astype(jnp.int32), padv])
    colp = jnp.concatenate([edge_index[1].astype(jnp.int32), padv])
    ridx = jnp.stack([colp, rowp]).reshape(2, 16, TILES, TB)
    colh = jnp.stack([colp + p * NP for p in range(8)]).reshape(
        8, 16, TILES, TB)

    c = jnp.cos(phase)
    s = jnp.sin(phase)
    ew = jax.nn.sigmoid(gate)
    hw = jax.nn.softmax(hop_weights)
    g1 = ew * hw[1]
    g2 = ew * hw[2]
    consts = jnp.stack([
        jnp.stack([c, -s, ew * hw[0] * c, -(ew * hw[0] * s), g1, g2]),
        jnp.stack([s, c, ew * hw[0] * s, ew * hw[0] * c, g1, g2]),
    ]).astype(f32)

    ones16h = jnp.ones((TB, 16), f32)
    zrow = jnp.zeros((RPS, HD), f32)
    zr16 = jnp.zeros((RPS, 16), f32)

    utab, w = _sc_mega(xrh, xih, colh, ridx, consts, ones16h, zrow, zr16)
    del utab
    wr = jnp.concatenate([w[0, :N], w[1, :N]], axis=1)
    wi = jnp.concatenate([w[0, NP:NP + N], w[1, NP:NP + N]], axis=1)

    out_r, out_i = _tc_call(wr, wi, x_real, x_imag, Wr, Wi,
                            br.reshape(1, D), bi.reshape(1, D))
    return (out_r, out_i)


# speedup vs baseline: 2.8889x; 1.0393x over previous
import functools

import jax
import jax.numpy as jnp
from jax import lax
from jax.experimental import pallas as pl
from jax.experimental.pallas import tpu as pltpu
from jax.experimental.pallas import tpu_sc as plsc

N = 10000
D = 128
E = 320000
HD = 64
NP = 10112
RPS = NP // 16
EP = 327680
TILES = 160
TB = 128
NCH = 5
TPC = TILES // NCH
DUMMY = NP - 1
CB = 64
CHUNKS = tuple((k * CB, CB) for k in range(9)) + ((9 * CB, RPS - 9 * CB),)


def _rsqrt16(x):
    i = lax.bitcast_convert_type(x, jnp.int32)
    i = jnp.int32(0x5F3759DF) - (i >> 1)
    y = lax.bitcast_convert_type(i, jnp.float32)
    for _ in range(3):
        y = y * (1.5 - 0.5 * x * y * y)
    return y


_mesh = plsc.VectorSubcoreMesh(core_axis_name="c", subcore_axis_name="s")


@functools.partial(
    pl.kernel,
    out_type=[
        jax.ShapeDtypeStruct((8 * NP, HD), jnp.float32),
        jax.ShapeDtypeStruct((2, 2 * NP, HD), jnp.float32),
    ],
    mesh=_mesh,
    compiler_params=pltpu.CompilerParams(use_tc_tiling_on_sc=False),
    scratch_types=[
        pltpu.VMEM_SHARED((NP, HD), jnp.float32),
        pltpu.VMEM_SHARED((NP, 16), jnp.float32),
        pltpu.VMEM((4, TB, HD), jnp.float32),
        pltpu.VMEM((CB, HD), jnp.float32),
        pltpu.VMEM((CB, HD), jnp.float32),
        pltpu.VMEM((CB, HD), jnp.float32),
        pltpu.VMEM((TPC, TB), jnp.int32),
        pltpu.VMEM((TPC, TB), jnp.int32),
        pltpu.VMEM((RPS, 16), jnp.float32),
        pltpu.VMEM((CB, 16), jnp.float32),
        pltpu.VMEM((TB, 16), jnp.float32),
        pltpu.VMEM((6, D), jnp.float32),
        pltpu.SemaphoreType.DMA((4,)),
        pltpu.SemaphoreType.DMA((4,)),
    ],
)
def _sc_mega(xrh, xih, colh, ridx, consts, ones16h, zrow, zr16,
             utab, w_all,
             acc, dacc, gbuf, bufA, bufB, bufC, colb, rowb,
             dis_l, dbuf, ones16, cbuf, gsem, ssem):
    cid = lax.axis_index("c")
    sid = lax.axis_index("s")
    base = sid * RPS
    coff = cid * NP

    pltpu.sync_copy(consts.at[cid], cbuf)
    pltpu.sync_copy(ones16h, ones16)
    pltpu.sync_copy(zrow, acc.at[pl.ds(base, RPS)])
    pltpu.sync_copy(zr16, dacc.at[pl.ds(base, RPS)])
    plsc.subcore_barrier()

    def dg_chunk(ch, _):
        pltpu.sync_copy(ridx.at[0, sid, pl.ds(ch * TPC, TPC)], rowb)

        def dg_body(i, _):
            r = i - 4

            @pl.when(r >= 0)
            def _():
                pltpu.make_async_copy(ones16, dacc.at[rowb.at[r]],
                                      ssem.at[lax.rem(r, 4)]).wait()

            @pl.when(i < TPC)
            def _():
                pltpu.async_copy(ones16, dacc.at[rowb.at[i]],
                                 ssem.at[lax.rem(i, 4)], add=True)
            return 0

        lax.fori_loop(0, TPC + 4, dg_body, 0)
        return 0

    lax.fori_loop(0, NCH, dg_chunk, 0)
    plsc.subcore_barrier()

    for off, sz in CHUNKS:
        pltpu.sync_copy(dacc.at[pl.ds(base + off, sz)],
                        dbuf.at[pl.ds(0, sz)])

        def rbody(r, _):
            v = dbuf[r] + 1.0
            dis_l[off + r] = _rsqrt16(v)
            return 0

        lax.fori_loop(0, sz, rbody, 0)

    for bf in range(2):
        for off, sz in CHUNKS:
            pltpu.sync_copy(xrh.at[bf, pl.ds(base + off, sz)],
                            bufA.at[pl.ds(0, sz)])
            pltpu.sync_copy(xih.at[bf, pl.ds(base + off, sz)],
                            bufB.at[pl.ds(0, sz)])

            def bbody(r, _):
                dd = dis_l[off + r][0]
                for g in range(4):
                    sl = pl.ds(g * 16, 16)
                    cl = pl.ds(bf * HD + g * 16, 16)
                    bufC[r, sl] = dd * (cbuf[0, cl] * bufA[r, sl]
                                        + cbuf[1, cl] * bufB[r, sl])
                return 0

            lax.fori_loop(0, sz, bbody, 0)
            pltpu.sync_copy(
                bufC.at[pl.ds(0, sz)],
                utab.at[pl.ds((2 * cid + bf) * NP + base + off, sz)])
    plsc.subcore_barrier()

    def phase_q(q, _):
        h = q // 2
        f = lax.rem(q, 2)
        gp = 4 * h + 2 * cid + f

        def chunk(ch, _):
            pltpu.sync_copy(ridx.at[1, sid, pl.ds(ch * TPC, TPC)], rowb)
            pltpu.sync_copy(colh.at[gp, sid, pl.ds(ch * TPC, TPC)], colb)

            def body(i, _):
                r = i - 4

                @pl.when(jnp.logical_and(r >= 0, r < TPC))
                def _():
                    pr = lax.rem(r, 4)
                    pltpu.make_async_copy(gbuf.at[pr], acc.at[rowb.at[r]],
                                          ssem.at[pr]).wait()

                @pl.when(i < TPC)
                def _():
                    pltpu.async_copy(utab.at[colb.at[i]],
                                     gbuf.at[lax.rem(i, 4)],
                                     gsem.at[lax.rem(i, 4)])

                j = i - 1

                @pl.when(jnp.logical_and(j >= 0, j < TPC))
                def _():
                    pj = lax.rem(j, 4)
                    pltpu.make_async_copy(utab.at[colb.at[j]],
                                          gbuf.at[pj], gsem.at[pj]).wait()
                    pltpu.async_copy(gbuf.at[pj], acc.at[rowb.at[j]],
                                     ssem.at[pj], add=True)

                return 0

            lax.fori_loop(0, TPC + 4, body, 0)
            return 0

        lax.fori_loop(0, NCH, chunk, 0)
        plsc.subcore_barrier()

        @pl.when(h == 0)
        def _():
            for off, sz in CHUNKS:
                pltpu.sync_copy(acc.at[pl.ds(base + off, sz)],
                                bufA.at[pl.ds(0, sz)])
                pltpu.sync_copy(
                    utab.at[pl.ds((2 * cid + f) * NP + base + off, sz)],
                    bufB.at[pl.ds(0, sz)])

                def ubody(r, _):
                    di = dis_l[off + r][0]
                    d2 = di * di
                    for g in range(4):
                        sl = pl.ds(g * 16, 16)
                        bufC[r, sl] = d2 * (bufA[r, sl] + bufB[r, sl])
                    return 0

                lax.fori_loop(0, sz, ubody, 0)
                pltpu.sync_copy(
                    bufC.at[pl.ds(0, sz)],
                    utab.at[pl.ds((4 + 2 * cid + f) * NP + base + off, sz)])
            pltpu.sync_copy(zrow, acc.at[pl.ds(base, RPS)])

        @pl.when(h == 1)
        def _():
            for off, sz in CHUNKS:
                pltpu.sync_copy(xrh.at[f, pl.ds(base + off, sz)],
                                bufA.at[pl.ds(0, sz)])
                pltpu.sync_copy(xih.at[f, pl.ds(base + off, sz)],
                                bufB.at[pl.ds(0, sz)])

                def fbody1(r, _):
                    for g in range(4):
                        sl = pl.ds(g * 16, 16)
                        cl = pl.ds(f * HD + g * 16, 16)
                        bufC[r, sl] = (cbuf[2, cl] * bufA[r, sl]
                                       + cbuf[3, cl] * bufB[r, sl])
                    return 0

                lax.fori_loop(0, sz, fbody1, 0)
                pltpu.sync_copy(
                    utab.at[pl.ds((4 + 2 * cid) * NP + f * NP
                                  + base + off, sz)],
                    bufA.at[pl.ds(0, sz)])
                pltpu.sync_copy(acc.at[pl.ds(base + off, sz)],
                                bufB.at[pl.ds(0, sz)])

                def fbody2(r, _):
                    dv = dis_l[off + r]
                    di = dv[0]
                    sd = _rsqrt16(dv * dv)[0]
                    for g in range(4):
                        sl = pl.ds(g * 16, 16)
                        cl = pl.ds(f * HD + g * 16, 16)
                        bufC[r, sl] = (bufC[r, sl]
                                       + sd * (cbuf[4, cl] * bufA[r, sl])
                                       + di * (cbuf[5, cl]
                                               * (bufA[r, sl] + bufB[r, sl])))
                    return 0

                lax.fori_loop(0, sz, fbody2, 0)
                pltpu.sync_copy(bufC.at[pl.ds(0, sz)],
                                w_all.at[f, pl.ds(coff + base + off, sz)])
            pltpu.sync_copy(zrow, acc.at[pl.ds(base, RPS)])

        plsc.subcore_barrier()
        return 0

    lax.fori_loop(0, 4, phase_q, 0)


def _tc_body(wr_ref, wi_ref, xr_ref, xi_ref, Wr_ref, Wi_ref, br_ref, bi_ref,
             or_ref, oi_ref):
    wr = wr_ref[...]
    wi = wi_ref[...]
    Wr = Wr_ref[...]
    Wi = Wi_ref[...]
    hp = jax.lax.Precision.HIGHEST
    or_ref[...] = (jnp.dot(wr, Wr, precision=hp,
                           preferred_element_type=jnp.float32)
                   - jnp.dot(wi, Wi, precision=hp,
                             preferred_element_type=jnp.float32)
                   + br_ref[...] + xr_ref[...])
    oi_ref[...] = (jnp.dot(wr, Wi, precision=hp,
                           preferred_element_type=jnp.float32)
                   + jnp.dot(wi, Wr, precision=hp,
                             preferred_element_type=jnp.float32)
                   + bi_ref[...] + xi_ref[...])


_TCB = 256
_tc_grid = (N + _TCB - 1) // _TCB


_row_spec = pl.BlockSpec((_TCB, D), lambda i: (i, 0))
_full_spec = pl.BlockSpec((D, D), lambda i: (0, 0))
_bias_spec = pl.BlockSpec((1, D), lambda i: (0, 0))

_tc_call = pl.pallas_call(
    _tc_body,
    grid=_tc_grid,
    in_specs=[_row_spec, _row_spec, _row_spec, _row_spec,
              _full_spec, _full_spec, _bias_spec, _bias_spec],
    out_specs=[_row_spec, _row_spec],
    out_shape=[jax.ShapeDtypeStruct((N, D), jnp.float32),
               jax.ShapeDtypeStruct((N, D), jnp.float32)],
)


def kernel(x_real, x_imag, edge_index, hop_weights, phase, gate, Wr, Wi, br, bi):
    f32 = jnp.float32
    xr = jnp.pad(x_real.astype(f32), ((0, NP - N), (0, 0)))
    xi = jnp.pad(x_imag.astype(f32), ((0, NP - N), (0, 0)))
    xrh = jnp.stack([xr[:, :HD], xr[:, HD:]])
    xih = jnp.stack([xi[:, :HD], xi[:, HD:]])

    padv = N + jnp.arange(EP - E, dtype=jnp.int32) % (NP - N)
    rowp = jnp.concatenate([edge_index[0].astype(jnp.int32), padv])
    colp = jnp.concatenate([edge_index[1].astype(jnp.int32), padv])
    ridx = jnp.stack([colp, rowp]).reshape(2, 16, TILES, TB)
    colh = jnp.stack([colp + p * NP for p in range(8)]).reshape(
        8, 16, TILES, TB)

    c = jnp.cos(phase)
    s = jnp.sin(phase)
    ew = jax.nn.sigmoid(gate)
    hw = jax.nn.softmax(hop_weights)
    g1 = ew * hw[1]
    g2 = ew * hw[2]
    consts = jnp.stack([
        jnp.stack([c, -s, ew * hw[0] * c, -(ew * hw[0] * s), g1, g2]),
        jnp.stack([s, c, ew * hw[0] * s, ew * hw[0] * c, g1, g2]),
    ]).astype(f32)

    ones16h = jnp.ones((TB, 16), f32)
    zrow = jnp.zeros((RPS, HD), f32)
    zr16 = jnp.zeros((RPS, 16), f32)

    utab, w = _sc_mega(xrh, xih, colh, ridx, consts, ones16h, zrow, zr16)
    del utab
    wr = jnp.concatenate([w[0, :N], w[1, :N]], axis=1)
    wi = jnp.concatenate([w[0, NP:NP + N], w[1, NP:NP + N]], axis=1)

    out_r, out_i = _tc_call(wr, wi, x_real, x_imag, Wr, Wi,
                            br.reshape(1, D), bi.reshape(1, D))
    return (out_r, out_i)
